# baseline (speedup 1.0000x reference)
import jax
import jax.numpy as jnp
from jax.experimental import pallas as pl


def _pos_body(row_ref, col_ref, out_ref):
    F = row_ref.shape[1]
    H = out_ref.shape[2]
    W = out_ref.shape[3]
    colT = col_ref[:W, :].T
    rowT = row_ref[:H, :].T
    out_ref[0, :F] = jnp.broadcast_to(colT[:, None, :], (F, H, W))
    out_ref[0, F:] = jnp.broadcast_to(rowT[:, :, None], (F, H, W))


def kernel(input, row_embed, col_embed):
    B, C, H, W = input.shape
    N, F = row_embed.shape
    out = pl.pallas_call(
        _pos_body,
        grid=(B,),
        in_specs=[
            pl.BlockSpec((N, F), lambda b: (0, 0)),
            pl.BlockSpec((N, F), lambda b: (0, 0)),
        ],
        out_specs=pl.BlockSpec((1, 2 * F, H, W), lambda b: (b, 0, 0, 0)),
        out_shape=jax.ShapeDtypeStruct((B, 2 * F, H, W), row_embed.dtype),
    )(row_embed, col_embed)
    return out


# speedup vs baseline: 2.5782x; 2.5782x over previous
import functools

import jax
import jax.numpy as jnp
from jax import lax
from jax.experimental import pallas as pl


def _pos_body(H, W, row_ref, col_ref, out_ref):
    F = row_ref.shape[1]
    HW = H * W
    lane_w = lax.broadcasted_iota(jnp.int32, (W, HW), 1)
    sub_w = lax.broadcasted_iota(jnp.int32, (W, HW), 0)
    tile_sel = (lane_w % W == sub_w).astype(jnp.float32)
    lane_h = lax.broadcasted_iota(jnp.int32, (H, HW), 1)
    sub_h = lax.broadcasted_iota(jnp.int32, (H, HW), 0)
    rep_sel = (lane_h // W == sub_h).astype(jnp.float32)
    dn = (((0,), (0,)), ((), ()))
    out_ref[0, :F] = lax.dot_general(
        col_ref[:W, :], tile_sel, dn, preferred_element_type=jnp.float32)
    out_ref[0, F:] = lax.dot_general(
        row_ref[:H, :], rep_sel, dn, preferred_element_type=jnp.float32)


def kernel(input, row_embed, col_embed):
    B, C, H, W = input.shape
    N, F = row_embed.shape
    out = pl.pallas_call(
        functools.partial(_pos_body, H, W),
        grid=(B,),
        in_specs=[
            pl.BlockSpec((N, F), lambda b: (0, 0)),
            pl.BlockSpec((N, F), lambda b: (0, 0)),
        ],
        out_specs=pl.BlockSpec((1, 2 * F, H * W), lambda b: (b, 0, 0)),
        out_shape=jax.ShapeDtypeStruct((B, 2 * F, H * W), row_embed.dtype),
    )(row_embed, col_embed)
    return out.reshape(B, 2 * F, H, W)


# speedup vs baseline: 2.7976x; 1.0851x over previous
import functools

import jax
import jax.numpy as jnp
from jax import lax
from jax.experimental import pallas as pl
from jax.experimental.pallas import tpu as pltpu


def _pos_body(B, H, W, row_ref, col_ref, out_ref, scratch, sem):
    F = row_ref.shape[1]
    HW = H * W
    lane_w = lax.broadcasted_iota(jnp.int32, (W, HW), 1)
    sub_w = lax.broadcasted_iota(jnp.int32, (W, HW), 0)
    tile_sel = (lane_w % W == sub_w).astype(jnp.float32)
    lane_h = lax.broadcasted_iota(jnp.int32, (H, HW), 1)
    sub_h = lax.broadcasted_iota(jnp.int32, (H, HW), 0)
    rep_sel = (lane_h // W == sub_h).astype(jnp.float32)
    dn = (((0,), (0,)), ((), ()))
    scratch[:F] = lax.dot_general(
        col_ref[:W, :], tile_sel, dn, preferred_element_type=jnp.float32)
    scratch[F:] = lax.dot_general(
        row_ref[:H, :], rep_sel, dn, preferred_element_type=jnp.float32)
    for b in range(B):
        pltpu.make_async_copy(scratch, out_ref.at[b], sem).start()
    for b in range(B):
        pltpu.make_async_copy(scratch, out_ref.at[b], sem).wait()


def kernel(input, row_embed, col_embed):
    B, C, H, W = input.shape
    N, F = row_embed.shape
    out = pl.pallas_call(
        functools.partial(_pos_body, B, H, W),
        in_specs=[
            pl.BlockSpec(memory_space=pltpu.MemorySpace.VMEM),
            pl.BlockSpec(memory_space=pltpu.MemorySpace.VMEM),
        ],
        out_specs=pl.BlockSpec(memory_space=pltpu.MemorySpace.HBM),
        out_shape=jax.ShapeDtypeStruct((B, 2 * F, H * W), row_embed.dtype),
        scratch_shapes=[
            pltpu.VMEM((2 * F, H * W), jnp.float32),
            pltpu.SemaphoreType.DMA,
        ],
    )(row_embed, col_embed)
    return out.reshape(B, 2 * F, H, W)
